# baseline (speedup 1.0000x reference)
import functools

import jax
import jax.numpy as jnp
from jax import lax
from jax.experimental import pallas as pl
from jax.experimental.pallas import tpu as pltpu
from jax.experimental.pallas import tpu_sc as plsc

N = 131072
C = 19
K = 1024
NCORES = 2
NSUB = 16
NW = NCORES * NSUB
RPT = N // NW
CHUNK = 256
NCHUNK = RPT // CHUNK
CK = C * K
TBL = 4 * CK
EPC = CHUNK * C
MAGIC = 55189
SHIFT = 20


def _sc_hist_body(out_hbm, tgt_hbm, hist_hbm, in_v, tgt_v, err_v, tbl_v):
    cid = lax.axis_index("c")
    sid = lax.axis_index("s")
    wid = sid * NCORES + cid
    iota = lax.iota(jnp.int32, 16)
    zeros16 = jnp.zeros((16,), jnp.float32)
    ones16 = jnp.ones((16,), jnp.float32)

    def zero_body(i, carry):
        tbl_v[pl.ds(i * 16, 16)] = zeros16
        return carry

    lax.fori_loop(0, TBL // 16, zero_body, 0, unroll=8)

    def chunk_body(ci, carry):
        row0 = wid * RPT + ci * CHUNK
        pltpu.sync_copy(out_hbm.at[wid, pl.ds(ci * EPC, EPC)], in_v)
        pltpu.sync_copy(tgt_hbm.at[pl.ds(row0, CHUNK)], tgt_v)

        def grp_body(gi, c2):
            r = gi * 16
            rows = r + iota
            base = rows * C
            vals = [plsc.load_gather(in_v, [base + c]) for c in range(C)]
            exps = [jnp.exp(v) for v in vals]
            s = exps[0]
            for c in range(1, C):
                s = s + exps[c]
            rcp = 1.0 / s
            tv = tgt_v[pl.ds(r, 16)]
            for c in range(C):
                p = exps[c] * rcp
                fg = tv == c
                err = jnp.where(fg, 1.0 - p, p)
                plsc.store_scatter(err_v, [base + c], err)
            return c2

        lax.fori_loop(0, CHUNK // 16, grp_body, 0)

        def scat_body(vi, c3):
            jb = vi * 16
            j = jb + iota
            row = lax.shift_right_logical(j * MAGIC, SHIFT)
            cc = j - row * C
            tr = plsc.load_gather(tgt_v, [row])
            fg = cc == tr
            e = err_v[pl.ds(jb, 16)]
            b = jnp.minimum((e * float(K)).astype(jnp.int32), K - 1)
            g = cc * K + b
            plsc.addupdate_scatter(tbl_v, [g], ones16)
            plsc.addupdate_scatter(tbl_v, [g + CK], e)
            plsc.addupdate_scatter(tbl_v, [g + 2 * CK], ones16, mask=fg)
            plsc.addupdate_scatter(tbl_v, [g + 3 * CK], e, mask=fg)
            return c3

        lax.fori_loop(0, EPC // 16, scat_body, 0, unroll=4)
        return carry

    lax.fori_loop(0, NCHUNK, chunk_body, 0)
    pltpu.sync_copy(tbl_v, hist_hbm.at[wid])


def _finish_body(hist_ref, out_ref):
    tot = jnp.sum(hist_ref[...], axis=0)
    bi = lax.broadcasted_iota(jnp.int32, (K, K), 0)
    bj = lax.broadcasted_iota(jnp.int32, (K, K), 1)
    m = (bi > bj).astype(jnp.float32)
    cum = jax.lax.dot_general(
        tot, m, (((1,), (0,)), ((), ())),
        preferred_element_type=jnp.float32)
    cntA = tot[0:C]
    sumA = tot[C:2 * C]
    cntP = tot[2 * C:3 * C]
    sumP = tot[3 * C:4 * C]
    cumA = cum[0:C]
    cumP = cum[2 * C:3 * C]
    cumB = cumA - cumP
    P = jnp.sum(cntP, axis=1, keepdims=True)
    denom = jnp.maximum(P + cumB, 0.5)
    lo = lax.broadcasted_iota(jnp.int32, (C, K), 1).astype(jnp.float32) \
        * (1.0 / K)
    j0w = cumA / denom * (1.0 / K)
    cntN = cntA - cntP
    sumN = sumA - sumP
    corr = (sumP - cntP * lo) / denom \
        + (sumN - cntN * lo) * (P - cumP) / (denom * denom)
    loss_c = jnp.sum(j0w + corr, axis=1)
    present = (P[:, 0] > 0).astype(jnp.float32)
    loss = jnp.sum(loss_c * present) / jnp.maximum(jnp.sum(present), 1.0)
    out_ref[...] = jnp.reshape(loss, (1, 1))


_sc_hist = pl.kernel(
    _sc_hist_body,
    out_type=jax.ShapeDtypeStruct((NW, TBL), jnp.float32),
    mesh=plsc.VectorSubcoreMesh(
        core_axis_name="c", subcore_axis_name="s",
        num_cores=NCORES, num_subcores=NSUB),
    compiler_params=pltpu.CompilerParams(needs_layout_passes=False),
    scratch_types=[
        pltpu.VMEM((EPC,), jnp.float32),
        pltpu.VMEM((CHUNK,), jnp.int32),
        pltpu.VMEM((EPC,), jnp.float32),
        pltpu.VMEM((TBL,), jnp.float32),
    ],
)

_finish = pl.pallas_call(
    _finish_body,
    out_shape=jax.ShapeDtypeStruct((1, 1), jnp.float32),
)


@jax.jit
def kernel(output, target):
    hist = _sc_hist(output.reshape(NW, RPT * C), target)
    loss = _finish(hist.reshape(NW, 4 * C, K))
    return loss.reshape(())


# speedup vs baseline: 1.5327x; 1.5327x over previous
import functools

import jax
import jax.numpy as jnp
from jax import lax
from jax.experimental import pallas as pl
from jax.experimental.pallas import tpu as pltpu
from jax.experimental.pallas import tpu_sc as plsc

N = 131072
C = 19
CPAD = 20
K = 64
NCORES = 2
NSUB = 16
NW = NCORES * NSUB
RPT = N // NW
CHUNK = 256
NCHUNK = RPT // CHUNK
EPC = CHUNK * C
SSTR = CPAD * K
LSTR = 4 * SSTR
TBL = 16 * LSTR


def _sc_hist_body(out_hbm, tgt_hbm, hist_hbm, in_v, tgt_v, tbl_v):
    cid = lax.axis_index("c")
    sid = lax.axis_index("s")
    wid = sid * NCORES + cid
    iota = lax.iota(jnp.int32, 16)
    zeros16 = jnp.zeros((16,), jnp.float32)
    ones16 = jnp.ones((16,), jnp.float32)
    lanebase = iota * LSTR

    def zero_body(i, carry):
        tbl_v[pl.ds(i * 16, 16)] = zeros16
        return carry

    lax.fori_loop(0, TBL // 16, zero_body, 0, unroll=8)

    def chunk_body(ci, carry):
        row0 = wid * RPT + ci * CHUNK
        pltpu.sync_copy(out_hbm.at[pl.ds(row0 * C, EPC)], in_v)
        pltpu.sync_copy(tgt_hbm.at[pl.ds(row0, CHUNK)], tgt_v)

        def grp_body(gi, c2):
            r = gi * 16
            rows = r + iota
            base = rows * C
            vals = [plsc.load_gather(in_v, [base + c]) for c in range(C)]
            exps = [jnp.exp(v) for v in vals]
            s = exps[0]
            for c in range(1, C):
                s = s + exps[c]
            rcp = 1.0 / s
            tv = tgt_v[pl.ds(r, 16)]
            errpos = zeros16
            for c in range(C):
                p = exps[c] * rcp
                fg = tv == c
                err = jnp.where(fg, 1.0 - p, p)
                errpos = jnp.where(fg, err, errpos)
                b = jnp.minimum((err * float(K)).astype(jnp.int32), K - 1)
                g = lanebase + (b + c * K)
                plsc.addupdate_scatter(tbl_v, [g], ones16)
                plsc.addupdate_scatter(tbl_v, [g + SSTR], err)
            bp = jnp.minimum((errpos * float(K)).astype(jnp.int32), K - 1)
            gp = lanebase + tv * K + (bp + 2 * SSTR)
            plsc.addupdate_scatter(tbl_v, [gp], ones16)
            plsc.addupdate_scatter(tbl_v, [gp + SSTR], errpos)
            return c2

        lax.fori_loop(0, CHUNK // 16, grp_body, 0)
        return carry

    lax.fori_loop(0, NCHUNK, chunk_body, 0)
    pltpu.sync_copy(tbl_v, hist_hbm.at[wid])


def _finish_body(hist_ref, out_ref):
    tot = jnp.sum(hist_ref[...], axis=0)
    t2 = tot.reshape(16 * 4 * CPAD * K // 128, 128)
    hi = lax.broadcasted_iota(jnp.int32, (LSTR // 128, 16 * LSTR // 128), 1)
    qi = lax.broadcasted_iota(jnp.int32, (LSTR // 128, 16 * LSTR // 128), 0)
    lmat = (hi % (LSTR // 128) == qi).astype(jnp.float32)
    t4 = jax.lax.dot_general(
        lmat, t2, (((1,), (0,)), ((), ())),
        preferred_element_type=jnp.float32)
    bi = lax.broadcasted_iota(jnp.int32, (128, 128), 0)
    bj = lax.broadcasted_iota(jnp.int32, (128, 128), 1)
    same = (bi // K) == (bj // K)
    m = ((bi > bj) & same).astype(jnp.float32)
    om = same.astype(jnp.float32)
    cum = jax.lax.dot_general(
        t4, m, (((1,), (0,)), ((), ())),
        preferred_element_type=jnp.float32)
    R = CPAD * K // 128
    cntA, sumA, cntP, sumP = (t4[i * R:(i + 1) * R] for i in range(4))
    cumA = cum[0:R]
    cumP = cum[2 * R:3 * R]
    P = jax.lax.dot_general(
        cntP, om, (((1,), (0,)), ((), ())),
        preferred_element_type=jnp.float32)
    cumB = cumA - cumP
    denom = jnp.maximum(P + cumB, 0.5)
    lo = (lax.broadcasted_iota(jnp.int32, (R, 128), 1) % K).astype(
        jnp.float32) * (1.0 / K)
    j0w = cumA / denom * (1.0 / K)
    cntN = cntA - cntP
    sumN = sumA - sumP
    corr = (sumP - cntP * lo) / denom \
        + (sumN - cntN * lo) * (P - cumP) / (denom * denom)
    present = (P > 0).astype(jnp.float32)
    loss_sum = jnp.sum((j0w + corr) * present)
    npresent = jnp.sum(present) * (1.0 / K)
    loss = loss_sum / jnp.maximum(npresent, 1.0)
    out_ref[...] = jnp.reshape(loss, (1, 1))


_sc_hist = pl.kernel(
    _sc_hist_body,
    out_type=jax.ShapeDtypeStruct((NW, TBL), jnp.float32),
    mesh=plsc.VectorSubcoreMesh(
        core_axis_name="c", subcore_axis_name="s",
        num_cores=NCORES, num_subcores=NSUB),
    compiler_params=pltpu.CompilerParams(needs_layout_passes=False),
    scratch_types=[
        pltpu.VMEM((EPC,), jnp.float32),
        pltpu.VMEM((CHUNK,), jnp.int32),
        pltpu.VMEM((TBL,), jnp.float32),
    ],
)

_finish = pl.pallas_call(
    _finish_body,
    out_shape=jax.ShapeDtypeStruct((1, 1), jnp.float32),
)


@jax.jit
def kernel(output, target):
    hist = _sc_hist(output.reshape(-1), target)
    loss = _finish(hist)
    return loss.reshape(())


# speedup vs baseline: 1.6073x; 1.0487x over previous
import functools

import jax
import jax.numpy as jnp
from jax import lax
from jax.experimental import pallas as pl
from jax.experimental.pallas import tpu as pltpu
from jax.experimental.pallas import tpu_sc as plsc

N = 131072
C = 19
CPAD = 20
K = 64
NCORES = 2
NSUB = 16
NW = NCORES * NSUB
RPT = N // NW
CHUNK = 512
NCHUNK = RPT // CHUNK
EPC = CHUNK * C
SSTR = CPAD * K
LSTR = 4 * SSTR
TBL = 16 * LSTR


def _sc_hist_body(out_hbm, tgt_hbm, hist_hbm, in_v, tgt_v, tbl_v):
    cid = lax.axis_index("c")
    sid = lax.axis_index("s")
    wid = sid * NCORES + cid
    iota = lax.iota(jnp.int32, 16)
    zeros16 = jnp.zeros((16,), jnp.float32)
    ones16 = jnp.ones((16,), jnp.float32)
    lanebase = iota * LSTR

    def zero_body(i, carry):
        tbl_v[pl.ds(i * 16, 16)] = zeros16
        return carry

    lax.fori_loop(0, TBL // 16, zero_body, 0, unroll=8)

    def chunk_body(ci, carry):
        row0 = wid * RPT + ci * CHUNK
        pltpu.sync_copy(out_hbm.at[pl.ds(row0 * C, EPC)], in_v)
        pltpu.sync_copy(tgt_hbm.at[pl.ds(row0, CHUNK)], tgt_v)

        def grp_body(gi, c2):
            r = gi * 16
            rows = r + iota
            base = rows * C
            vals = [plsc.load_gather(in_v, [base + c]) for c in range(C)]
            exps = [jnp.exp(v) for v in vals]
            s = exps[0]
            for c in range(1, C):
                s = s + exps[c]
            rcp = 1.0 / s
            tv = tgt_v[pl.ds(r, 16)]
            errpos = zeros16
            for c in range(C):
                p = exps[c] * rcp
                fg = tv == c
                err = jnp.where(fg, 1.0 - p, p)
                errpos = jnp.where(fg, err, errpos)
                b = jnp.minimum((err * float(K)).astype(jnp.int32), K - 1)
                g = lanebase + (b + c * K)
                plsc.addupdate_scatter(tbl_v, [g], ones16)
                plsc.addupdate_scatter(tbl_v, [g + SSTR], err)
            bp = jnp.minimum((errpos * float(K)).astype(jnp.int32), K - 1)
            gp = lanebase + tv * K + (bp + 2 * SSTR)
            plsc.addupdate_scatter(tbl_v, [gp], ones16)
            plsc.addupdate_scatter(tbl_v, [gp + SSTR], errpos)
            return c2

        lax.fori_loop(0, CHUNK // 16, grp_body, 0)
        return carry

    lax.fori_loop(0, NCHUNK, chunk_body, 0)
    pltpu.sync_copy(tbl_v, hist_hbm.at[wid])


def _finish_body(hist_ref, out_ref):
    tot = jnp.sum(hist_ref[...], axis=0)
    t2 = tot.reshape(16 * 4 * CPAD * K // 128, 128)
    hi = lax.broadcasted_iota(jnp.int32, (LSTR // 128, 16 * LSTR // 128), 1)
    qi = lax.broadcasted_iota(jnp.int32, (LSTR // 128, 16 * LSTR // 128), 0)
    lmat = (hi % (LSTR // 128) == qi).astype(jnp.float32)
    t4 = jax.lax.dot_general(
        lmat, t2, (((1,), (0,)), ((), ())),
        preferred_element_type=jnp.float32)
    bi = lax.broadcasted_iota(jnp.int32, (128, 128), 0)
    bj = lax.broadcasted_iota(jnp.int32, (128, 128), 1)
    same = (bi // K) == (bj // K)
    m = ((bi > bj) & same).astype(jnp.float32)
    om = same.astype(jnp.float32)
    cum = jax.lax.dot_general(
        t4, m, (((1,), (0,)), ((), ())),
        preferred_element_type=jnp.float32)
    R = CPAD * K // 128
    cntA, sumA, cntP, sumP = (t4[i * R:(i + 1) * R] for i in range(4))
    cumA = cum[0:R]
    cumP = cum[2 * R:3 * R]
    P = jax.lax.dot_general(
        cntP, om, (((1,), (0,)), ((), ())),
        preferred_element_type=jnp.float32)
    cumB = cumA - cumP
    denom = jnp.maximum(P + cumB, 0.5)
    lo = (lax.broadcasted_iota(jnp.int32, (R, 128), 1) % K).astype(
        jnp.float32) * (1.0 / K)
    j0w = cumA / denom * (1.0 / K)
    cntN = cntA - cntP
    sumN = sumA - sumP
    corr = (sumP - cntP * lo) / denom \
        + (sumN - cntN * lo) * (P - cumP) / (denom * denom)
    present = (P > 0).astype(jnp.float32)
    loss_sum = jnp.sum((j0w + corr) * present)
    npresent = jnp.sum(present) * (1.0 / K)
    loss = loss_sum / jnp.maximum(npresent, 1.0)
    out_ref[...] = jnp.reshape(loss, (1, 1))


_sc_hist = pl.kernel(
    _sc_hist_body,
    out_type=jax.ShapeDtypeStruct((NW, TBL), jnp.float32),
    mesh=plsc.VectorSubcoreMesh(
        core_axis_name="c", subcore_axis_name="s",
        num_cores=NCORES, num_subcores=NSUB),
    compiler_params=pltpu.CompilerParams(needs_layout_passes=False),
    scratch_types=[
        pltpu.VMEM((EPC,), jnp.float32),
        pltpu.VMEM((CHUNK,), jnp.int32),
        pltpu.VMEM((TBL,), jnp.float32),
    ],
)

_finish = pl.pallas_call(
    _finish_body,
    out_shape=jax.ShapeDtypeStruct((1, 1), jnp.float32),
)


@jax.jit
def kernel(output, target):
    hist = _sc_hist(output.reshape(-1), target)
    loss = _finish(hist)
    return loss.reshape(())


# speedup vs baseline: 1.6510x; 1.0272x over previous
import functools

import jax
import jax.numpy as jnp
from jax import lax
from jax.experimental import pallas as pl
from jax.experimental.pallas import tpu as pltpu
from jax.experimental.pallas import tpu_sc as plsc

N = 131072
C = 19
CPAD = 20
K = 64
NCORES = 2
NSUB = 16
NW = NCORES * NSUB
RPT = N // NW
CHUNK = 1024
NCHUNK = RPT // CHUNK
EPC = CHUNK * C
SSTR = CPAD * K
LSTR = 4 * SSTR
TBL = 16 * LSTR


def _sc_hist_body(out_hbm, tgt_hbm, hist_hbm, in_v, tgt_v, tbl_v):
    cid = lax.axis_index("c")
    sid = lax.axis_index("s")
    wid = sid * NCORES + cid
    iota = lax.iota(jnp.int32, 16)
    zeros16 = jnp.zeros((16,), jnp.float32)
    ones16 = jnp.ones((16,), jnp.float32)
    lanebase = iota * LSTR

    def zero_body(i, carry):
        tbl_v[pl.ds(i * 16, 16)] = zeros16
        return carry

    lax.fori_loop(0, TBL // 16, zero_body, 0, unroll=8)

    def chunk_body(ci, carry):
        row0 = wid * RPT + ci * CHUNK
        pltpu.sync_copy(out_hbm.at[pl.ds(row0 * C, EPC)], in_v)
        pltpu.sync_copy(tgt_hbm.at[pl.ds(row0, CHUNK)], tgt_v)

        def grp_body(gi, c2):
            r = gi * 16
            rows = r + iota
            base = rows * C
            vals = [plsc.load_gather(in_v, [base + c]) for c in range(C)]
            exps = [jnp.exp(v) for v in vals]
            s = exps[0]
            for c in range(1, C):
                s = s + exps[c]
            rcp = 1.0 / s
            tv = tgt_v[pl.ds(r, 16)]
            errpos = zeros16
            for c in range(C):
                p = exps[c] * rcp
                fg = tv == c
                err = jnp.where(fg, 1.0 - p, p)
                errpos = jnp.where(fg, err, errpos)
                b = jnp.minimum((err * float(K)).astype(jnp.int32), K - 1)
                g = lanebase + (b + c * K)
                plsc.addupdate_scatter(tbl_v, [g], ones16)
                plsc.addupdate_scatter(tbl_v, [g + SSTR], err)
            bp = jnp.minimum((errpos * float(K)).astype(jnp.int32), K - 1)
            gp = lanebase + tv * K + (bp + 2 * SSTR)
            plsc.addupdate_scatter(tbl_v, [gp], ones16)
            plsc.addupdate_scatter(tbl_v, [gp + SSTR], errpos)
            return c2

        lax.fori_loop(0, CHUNK // 16, grp_body, 0)
        return carry

    lax.fori_loop(0, NCHUNK, chunk_body, 0)
    pltpu.sync_copy(tbl_v, hist_hbm.at[wid])


def _finish_body(hist_ref, out_ref):
    tot = jnp.sum(hist_ref[...], axis=0)
    t2 = tot.reshape(16 * 4 * CPAD * K // 128, 128)
    hi = lax.broadcasted_iota(jnp.int32, (LSTR // 128, 16 * LSTR // 128), 1)
    qi = lax.broadcasted_iota(jnp.int32, (LSTR // 128, 16 * LSTR // 128), 0)
    lmat = (hi % (LSTR // 128) == qi).astype(jnp.float32)
    t4 = jax.lax.dot_general(
        lmat, t2, (((1,), (0,)), ((), ())),
        preferred_element_type=jnp.float32)
    bi = lax.broadcasted_iota(jnp.int32, (128, 128), 0)
    bj = lax.broadcasted_iota(jnp.int32, (128, 128), 1)
    same = (bi // K) == (bj // K)
    m = ((bi > bj) & same).astype(jnp.float32)
    om = same.astype(jnp.float32)
    cum = jax.lax.dot_general(
        t4, m, (((1,), (0,)), ((), ())),
        preferred_element_type=jnp.float32)
    R = CPAD * K // 128
    cntA, sumA, cntP, sumP = (t4[i * R:(i + 1) * R] for i in range(4))
    cumA = cum[0:R]
    cumP = cum[2 * R:3 * R]
    P = jax.lax.dot_general(
        cntP, om, (((1,), (0,)), ((), ())),
        preferred_element_type=jnp.float32)
    cumB = cumA - cumP
    denom = jnp.maximum(P + cumB, 0.5)
    lo = (lax.broadcasted_iota(jnp.int32, (R, 128), 1) % K).astype(
        jnp.float32) * (1.0 / K)
    j0w = cumA / denom * (1.0 / K)
    cntN = cntA - cntP
    sumN = sumA - sumP
    corr = (sumP - cntP * lo) / denom \
        + (sumN - cntN * lo) * (P - cumP) / (denom * denom)
    present = (P > 0).astype(jnp.float32)
    loss_sum = jnp.sum((j0w + corr) * present)
    npresent = jnp.sum(present) * (1.0 / K)
    loss = loss_sum / jnp.maximum(npresent, 1.0)
    out_ref[...] = jnp.reshape(loss, (1, 1))


_sc_hist = pl.kernel(
    _sc_hist_body,
    out_type=jax.ShapeDtypeStruct((NW, TBL), jnp.float32),
    mesh=plsc.VectorSubcoreMesh(
        core_axis_name="c", subcore_axis_name="s",
        num_cores=NCORES, num_subcores=NSUB),
    compiler_params=pltpu.CompilerParams(needs_layout_passes=False),
    scratch_types=[
        pltpu.VMEM((EPC,), jnp.float32),
        pltpu.VMEM((CHUNK,), jnp.int32),
        pltpu.VMEM((TBL,), jnp.float32),
    ],
)

_finish = pl.pallas_call(
    _finish_body,
    out_shape=jax.ShapeDtypeStruct((1, 1), jnp.float32),
)


@jax.jit
def kernel(output, target):
    hist = _sc_hist(output.reshape(-1), target)
    loss = _finish(hist)
    return loss.reshape(())


# speedup vs baseline: 1.6745x; 1.0142x over previous
import functools

import jax
import jax.numpy as jnp
from jax import lax
from jax.experimental import pallas as pl
from jax.experimental.pallas import tpu as pltpu
from jax.experimental.pallas import tpu_sc as plsc

N = 131072
C = 19
CPAD = 20
K = 64
NCORES = 2
NSUB = 16
NW = NCORES * NSUB
RPT = N // NW
CHUNK = 2048
NCHUNK = RPT // CHUNK
EPC = CHUNK * C
SSTR = CPAD * K
LSTR = 4 * SSTR
TBL = 16 * LSTR


def _sc_hist_body(out_hbm, tgt_hbm, hist_hbm, in_v, tgt_v, tbl_v):
    cid = lax.axis_index("c")
    sid = lax.axis_index("s")
    wid = sid * NCORES + cid
    iota = lax.iota(jnp.int32, 16)
    zeros16 = jnp.zeros((16,), jnp.float32)
    ones16 = jnp.ones((16,), jnp.float32)
    lanebase = iota * LSTR

    def zero_body(i, carry):
        tbl_v[pl.ds(i * 16, 16)] = zeros16
        return carry

    lax.fori_loop(0, TBL // 16, zero_body, 0, unroll=8)

    def chunk_body(ci, carry):
        row0 = wid * RPT + ci * CHUNK
        pltpu.sync_copy(out_hbm.at[pl.ds(row0 * C, EPC)], in_v)
        pltpu.sync_copy(tgt_hbm.at[pl.ds(row0, CHUNK)], tgt_v)

        def grp_body(gi, c2):
            r = gi * 16
            rows = r + iota
            base = rows * C
            vals = [plsc.load_gather(in_v, [base + c]) for c in range(C)]
            exps = [jnp.exp(v) for v in vals]
            s = exps[0]
            for c in range(1, C):
                s = s + exps[c]
            rcp = 1.0 / s
            tv = tgt_v[pl.ds(r, 16)]
            errpos = zeros16
            for c in range(C):
                p = exps[c] * rcp
                fg = tv == c
                err = jnp.where(fg, 1.0 - p, p)
                errpos = jnp.where(fg, err, errpos)
                b = jnp.minimum((err * float(K)).astype(jnp.int32), K - 1)
                g = lanebase + (b + c * K)
                plsc.addupdate_scatter(tbl_v, [g], ones16)
                plsc.addupdate_scatter(tbl_v, [g + SSTR], err)
            bp = jnp.minimum((errpos * float(K)).astype(jnp.int32), K - 1)
            gp = lanebase + tv * K + (bp + 2 * SSTR)
            plsc.addupdate_scatter(tbl_v, [gp], ones16)
            plsc.addupdate_scatter(tbl_v, [gp + SSTR], errpos)
            return c2

        lax.fori_loop(0, CHUNK // 16, grp_body, 0)
        return carry

    lax.fori_loop(0, NCHUNK, chunk_body, 0)
    pltpu.sync_copy(tbl_v, hist_hbm.at[wid])


def _finish_body(hist_ref, out_ref):
    tot = jnp.sum(hist_ref[...], axis=0)
    t2 = tot.reshape(16 * 4 * CPAD * K // 128, 128)
    hi = lax.broadcasted_iota(jnp.int32, (LSTR // 128, 16 * LSTR // 128), 1)
    qi = lax.broadcasted_iota(jnp.int32, (LSTR // 128, 16 * LSTR // 128), 0)
    lmat = (hi % (LSTR // 128) == qi).astype(jnp.float32)
    t4 = jax.lax.dot_general(
        lmat, t2, (((1,), (0,)), ((), ())),
        preferred_element_type=jnp.float32)
    bi = lax.broadcasted_iota(jnp.int32, (128, 128), 0)
    bj = lax.broadcasted_iota(jnp.int32, (128, 128), 1)
    same = (bi // K) == (bj // K)
    m = ((bi > bj) & same).astype(jnp.float32)
    om = same.astype(jnp.float32)
    cum = jax.lax.dot_general(
        t4, m, (((1,), (0,)), ((), ())),
        preferred_element_type=jnp.float32)
    R = CPAD * K // 128
    cntA, sumA, cntP, sumP = (t4[i * R:(i + 1) * R] for i in range(4))
    cumA = cum[0:R]
    cumP = cum[2 * R:3 * R]
    P = jax.lax.dot_general(
        cntP, om, (((1,), (0,)), ((), ())),
        preferred_element_type=jnp.float32)
    cumB = cumA - cumP
    denom = jnp.maximum(P + cumB, 0.5)
    lo = (lax.broadcasted_iota(jnp.int32, (R, 128), 1) % K).astype(
        jnp.float32) * (1.0 / K)
    j0w = cumA / denom * (1.0 / K)
    cntN = cntA - cntP
    sumN = sumA - sumP
    corr = (sumP - cntP * lo) / denom \
        + (sumN - cntN * lo) * (P - cumP) / (denom * denom)
    present = (P > 0).astype(jnp.float32)
    loss_sum = jnp.sum((j0w + corr) * present)
    npresent = jnp.sum(present) * (1.0 / K)
    loss = loss_sum / jnp.maximum(npresent, 1.0)
    out_ref[...] = jnp.reshape(loss, (1, 1))


_sc_hist = pl.kernel(
    _sc_hist_body,
    out_type=jax.ShapeDtypeStruct((NW, TBL), jnp.float32),
    mesh=plsc.VectorSubcoreMesh(
        core_axis_name="c", subcore_axis_name="s",
        num_cores=NCORES, num_subcores=NSUB),
    compiler_params=pltpu.CompilerParams(needs_layout_passes=False),
    scratch_types=[
        pltpu.VMEM((EPC,), jnp.float32),
        pltpu.VMEM((CHUNK,), jnp.int32),
        pltpu.VMEM((TBL,), jnp.float32),
    ],
)

_finish = pl.pallas_call(
    _finish_body,
    out_shape=jax.ShapeDtypeStruct((1, 1), jnp.float32),
)


@jax.jit
def kernel(output, target):
    hist = _sc_hist(output.reshape(-1), target)
    loss = _finish(hist)
    return loss.reshape(())
